# baseline (speedup 1.0000x reference)
import functools

import jax
import jax.numpy as jnp
from jax import lax
from jax.experimental import pallas as pl
from jax.experimental.pallas import tpu as pltpu
from jax.experimental.pallas import tpu_sc as plsc

BATCH = 16384
D = 64
NC = 2
NS = 16
NW = NC * NS
BPW = BATCH // NW


def _sc_gather6(lhs_idx, rel_idx, rhs_idx, ent_embeds, rel_embeds,
                ent_transfer, rel_transfer):
    mesh = plsc.VectorSubcoreMesh(core_axis_name="c", subcore_axis_name="s")
    out_t = jax.ShapeDtypeStruct((BATCH, D), jnp.float32)

    @functools.partial(
        pl.kernel,
        mesh=mesh,
        compiler_params=pltpu.CompilerParams(use_tc_tiling_on_sc=False),
        out_type=[out_t] * 6,
        scratch_types=[
            pltpu.VMEM((BPW,), jnp.int32),
            pltpu.VMEM((BPW, D), jnp.float32),
            pltpu.VMEM((BPW, D), jnp.float32),
            pltpu.SemaphoreType.DMA,
            pltpu.SemaphoreType.DMA,
        ],
    )
    def k(lhs_hbm, rel_hbm, rhs_hbm, ee_hbm, re_hbm, et_hbm, rt_hbm,
          o_re, o_rt, o_le, o_lt, o_rhe, o_rht,
          idx_v, buf0, buf1, sem0, sem1):
        wid = lax.axis_index("s") * NC + lax.axis_index("c")
        base = wid * BPW

        def pair(idx_hbm, tbl0, out0, tbl1, out1):
            pltpu.sync_copy(idx_hbm.at[pl.ds(base, BPW)], idx_v)
            cp0 = pltpu.async_copy(tbl0.at[idx_v], buf0, sem0)
            cp1 = pltpu.async_copy(tbl1.at[idx_v], buf1, sem1)
            cp0.wait()
            pltpu.sync_copy(buf0, out0.at[pl.ds(base, BPW)])
            cp1.wait()
            pltpu.sync_copy(buf1, out1.at[pl.ds(base, BPW)])

        pair(rel_hbm, re_hbm, o_re, rt_hbm, o_rt)
        pair(lhs_hbm, ee_hbm, o_le, et_hbm, o_lt)
        pair(rhs_hbm, ee_hbm, o_rhe, et_hbm, o_rht)

    return k(lhs_idx, rel_idx, rhs_idx, ent_embeds, rel_embeds,
             ent_transfer, rel_transfer)


_BLK = 2048


def _tc_energy_body(re_r, rt_r, le_r, lt_r, rhe_r, rht_r, out_r):
    def scaled(ref):
        x = ref[...]
        n = jnp.sqrt(jnp.sum(x * x, axis=1, keepdims=True))
        return x * jnp.where(n > 1.0, 1.0 / (n + 1e-7), 1.0)

    re_s = scaled(re_r)
    rt_s = scaled(rt_r)
    le_s = scaled(le_r)
    lt_s = scaled(lt_r)
    rhe_s = scaled(rhe_r)
    rht_s = scaled(rht_r)
    lhs = le_s + jnp.sum(le_s * lt_s, axis=1, keepdims=True) * rt_s
    rhs = rhe_s + jnp.sum(rhe_s * rht_s, axis=1, keepdims=True) * rt_s
    diff = lhs + re_s - rhs
    out_r[...] = jnp.sqrt(jnp.sum(diff * diff, axis=1) + 1e-12)


def _tc_energy(re, rt, le, lt, rhe, rht):
    in_spec = pl.BlockSpec((_BLK, D), lambda i: (i, 0))
    out_spec = pl.BlockSpec((_BLK,), lambda i: (i,))
    return pl.pallas_call(
        _tc_energy_body,
        grid=(BATCH // _BLK,),
        in_specs=[in_spec] * 6,
        out_specs=out_spec,
        out_shape=jax.ShapeDtypeStruct((BATCH,), jnp.float32),
    )(re, rt, le, lt, rhe, rht)


def kernel(triplets, ent_embeds, rel_embeds, ent_transfer, rel_transfer):
    lhs_idx = triplets[:, 0]
    rel_idx = triplets[:, 1]
    rhs_idx = triplets[:, 2]
    o_re, o_rt, o_le, o_lt, o_rhe, o_rht = _sc_gather6(
        lhs_idx, rel_idx, rhs_idx, ent_embeds, rel_embeds,
        ent_transfer, rel_transfer)
    return _tc_energy(o_re, o_rt, o_le, o_lt, o_rhe, o_rht)


# speedup vs baseline: 1.2973x; 1.2973x over previous
import functools

import jax
import jax.numpy as jnp
from jax import lax
from jax.experimental import pallas as pl
from jax.experimental.pallas import tpu as pltpu
from jax.experimental.pallas import tpu_sc as plsc

BATCH = 16384
D = 64
NC = 2
NS = 16
NW = NC * NS
BPW = BATCH // NW
CHUNK = 256
NRING = 8


def _sc_gather6(lhs_idx, rel_idx, rhs_idx, ent_embeds, rel_embeds,
                ent_transfer, rel_transfer):
    mesh = plsc.VectorSubcoreMesh(core_axis_name="c", subcore_axis_name="s")
    out_t = jax.ShapeDtypeStruct((BATCH, D), jnp.float32)

    @functools.partial(
        pl.kernel,
        mesh=mesh,
        out_type=[out_t] * 6,
        scratch_types=[
            pltpu.VMEM((BPW,), jnp.int32),
            pltpu.VMEM((CHUNK, D), jnp.float32),
            pltpu.VMEM((CHUNK, D), jnp.float32),
            pltpu.SemaphoreType.DMA((16,)),
            pltpu.SemaphoreType.DMA,
        ],
    )
    def k(lhs_hbm, rel_hbm, rhs_hbm, ee_hbm, re_hbm, et_hbm, rt_hbm,
          o_re, o_rt, o_le, o_lt, o_rhe, o_rht,
          idx_v, buf0, buf1, ring, wsem):
        wid = lax.axis_index("s") * NC + lax.axis_index("c")
        base = wid * BPW

        def fill(tbl, buf, off):
            def body(g, _):
                v = idx_v[pl.ds(off + g * 16, 16)]
                b = g * 16

                @pl.when(g >= 1)
                def _():
                    for j in range(16):
                        pltpu.make_async_copy(
                            tbl.at[0], buf.at[b - 16 + j], ring.at[j]).wait()

                for j in range(16):
                    pltpu.make_async_copy(
                        tbl.at[v[j]], buf.at[b + j], ring.at[j]).start()
                return 0

            lax.fori_loop(0, CHUNK // 16, body, 0)
            for j in range(16):
                pltpu.make_async_copy(
                    tbl.at[0], buf.at[CHUNK - 16 + j], ring.at[j]).wait()

        def job(idx_hbm, tbl, out, load_idx):
            if load_idx:
                pltpu.sync_copy(idx_hbm.at[pl.ds(base, BPW)], idx_v)
            fill(tbl, buf0, 0)
            cp0 = pltpu.make_async_copy(
                buf0, out.at[pl.ds(base, CHUNK)], wsem)
            cp0.start()
            fill(tbl, buf1, CHUNK)
            cp0.wait()
            cp1 = pltpu.make_async_copy(
                buf1, out.at[pl.ds(base + CHUNK, CHUNK)], wsem)
            cp1.start()
            cp1.wait()

        job(rel_hbm, re_hbm, o_re, True)
        job(rel_hbm, rt_hbm, o_rt, False)
        job(lhs_hbm, ee_hbm, o_le, True)
        job(lhs_hbm, et_hbm, o_lt, False)
        job(rhs_hbm, ee_hbm, o_rhe, True)
        job(rhs_hbm, et_hbm, o_rht, False)

    return k(lhs_idx, rel_idx, rhs_idx, ent_embeds, rel_embeds,
             ent_transfer, rel_transfer)


_BLK = 2048


def _tc_energy_body(re_r, rt_r, le_r, lt_r, rhe_r, rht_r, out_r):
    def scaled(ref):
        x = ref[...]
        n = jnp.sqrt(jnp.sum(x * x, axis=1, keepdims=True))
        return x * jnp.where(n > 1.0, 1.0 / (n + 1e-7), 1.0)

    re_s = scaled(re_r)
    rt_s = scaled(rt_r)
    le_s = scaled(le_r)
    lt_s = scaled(lt_r)
    rhe_s = scaled(rhe_r)
    rht_s = scaled(rht_r)
    lhs = le_s + jnp.sum(le_s * lt_s, axis=1, keepdims=True) * rt_s
    rhs = rhe_s + jnp.sum(rhe_s * rht_s, axis=1, keepdims=True) * rt_s
    diff = lhs + re_s - rhs
    out_r[...] = jnp.sqrt(jnp.sum(diff * diff, axis=1) + 1e-12)


def _tc_energy(re, rt, le, lt, rhe, rht):
    in_spec = pl.BlockSpec((_BLK, D), lambda i: (i, 0))
    out_spec = pl.BlockSpec((_BLK,), lambda i: (i,))
    return pl.pallas_call(
        _tc_energy_body,
        grid=(BATCH // _BLK,),
        in_specs=[in_spec] * 6,
        out_specs=out_spec,
        out_shape=jax.ShapeDtypeStruct((BATCH,), jnp.float32),
    )(re, rt, le, lt, rhe, rht)


def kernel(triplets, ent_embeds, rel_embeds, ent_transfer, rel_transfer):
    lhs_idx = triplets[:, 0]
    rel_idx = triplets[:, 1]
    rhs_idx = triplets[:, 2]
    o_re, o_rt, o_le, o_lt, o_rhe, o_rht = _sc_gather6(
        lhs_idx, rel_idx, rhs_idx, ent_embeds, rel_embeds,
        ent_transfer, rel_transfer)
    return _tc_energy(o_re, o_rt, o_le, o_lt, o_rhe, o_rht)


# speedup vs baseline: 2.7882x; 2.1493x over previous
import functools

import jax
import jax.numpy as jnp
from jax import lax
from jax.experimental import pallas as pl
from jax.experimental.pallas import tpu as pltpu
from jax.experimental.pallas import tpu_sc as plsc

NUM = 1000000
SPLIT = 499712
OUTROWS = NUM - SPLIT
BATCH = 16384
D = 64
NC = 2
NS = 16
NW = NC * NS
BPW = BATCH // NW
CH = BPW // 2
TBK = 4096
_LOBLK = SPLIT // TBK


def _tc_repack_body(*refs):
    ins, outs = refs[:8], refs[8:]
    for t in range(4):
        lo, hi, o = ins[2 * t], ins[2 * t + 1], outs[t]
        o[...] = jnp.concatenate([lo[...], hi[...]], axis=0).T


def _tc_repack(t1, t2, t3, t4):
    nblk = OUTROWS // TBK + 1
    in_lo = pl.BlockSpec((D, TBK), lambda i: (0, i))
    in_hi = pl.BlockSpec((D, TBK), lambda i: (0, i + _LOBLK))
    out_s = pl.BlockSpec((TBK, 2 * D), lambda i: (i, 0))
    return pl.pallas_call(
        _tc_repack_body,
        grid=(nblk,),
        in_specs=[in_lo, in_hi] * 4,
        out_specs=[out_s] * 4,
        out_shape=[jax.ShapeDtypeStruct((OUTROWS, 2 * D), jnp.float32)] * 4,
    )(t1, t1, t2, t2, t3, t3, t4, t4)


def _sc_gather6(lhs_idx, rel_idx, rhs_idx, ee, re, et, rt):
    mesh = plsc.VectorSubcoreMesh(core_axis_name="c", subcore_axis_name="s")
    out_t = jax.ShapeDtypeStruct((BATCH, 2 * D), jnp.float32)

    @functools.partial(
        pl.kernel,
        mesh=mesh,
        out_type=[out_t] * 6,
        scratch_types=[
            pltpu.VMEM((CH,), jnp.int32),
            pltpu.VMEM((CH,), jnp.int32),
            pltpu.VMEM((CH, 2 * D), jnp.float32),
            pltpu.VMEM((CH, 2 * D), jnp.float32),
            pltpu.SemaphoreType.DMA,
            pltpu.SemaphoreType.DMA,
            pltpu.SemaphoreType.DMA,
            pltpu.SemaphoreType.DMA,
        ],
    )
    def k(lhs_hbm, rel_hbm, rhs_hbm, ee_hbm, re_hbm, et_hbm, rt_hbm,
          o_re, o_rt, o_le, o_lt, o_rhe, o_rht,
          idx0, idx1, buf0, buf1, g0, g1, w0, w1):
        wid = lax.axis_index("s") * NC + lax.axis_index("c")
        base = wid * BPW

        def job(idx_hbm, tbl, out, load_idx):
            if load_idx:
                pltpu.sync_copy(idx_hbm.at[pl.ds(base, CH)], idx0)
                pltpu.sync_copy(idx_hbm.at[pl.ds(base + CH, CH)], idx1)
            cp0 = pltpu.async_copy(tbl.at[idx0], buf0, g0)
            cp1 = pltpu.async_copy(tbl.at[idx1], buf1, g1)
            cp0.wait()
            wr0 = pltpu.make_async_copy(buf0, out.at[pl.ds(base, CH)], w0)
            wr0.start()
            cp1.wait()
            wr1 = pltpu.make_async_copy(
                buf1, out.at[pl.ds(base + CH, CH)], w1)
            wr1.start()
            wr0.wait()
            wr1.wait()

        job(rel_hbm, re_hbm, o_re, True)
        job(rel_hbm, rt_hbm, o_rt, False)
        job(lhs_hbm, ee_hbm, o_le, True)
        job(lhs_hbm, et_hbm, o_lt, False)
        job(rhs_hbm, ee_hbm, o_rhe, True)
        job(rhs_hbm, et_hbm, o_rht, False)

    return k(lhs_idx, rel_idx, rhs_idx, ee, re, et, rt)


_BLK = 2048


def _tc_energy_body(re_r, rt_r, le_r, lt_r, rhe_r, rht_r,
                    li_r, ri_r, hi_r, out_r):
    def half(ref, sel_r):
        x = ref[...]
        sel = sel_r[...] >= SPLIT
        return jnp.where(sel, x[:, D:2 * D], x[:, 0:D])

    def scaled(x):
        n = jnp.sqrt(jnp.sum(x * x, axis=1, keepdims=True))
        return x * jnp.where(n > 1.0, 1.0 / (n + 1e-7), 1.0)

    re_s = scaled(half(re_r, ri_r))
    rt_s = scaled(half(rt_r, ri_r))
    le_s = scaled(half(le_r, li_r))
    lt_s = scaled(half(lt_r, li_r))
    rhe_s = scaled(half(rhe_r, hi_r))
    rht_s = scaled(half(rht_r, hi_r))
    lhs = le_s + jnp.sum(le_s * lt_s, axis=1, keepdims=True) * rt_s
    rhs = rhe_s + jnp.sum(rhe_s * rht_s, axis=1, keepdims=True) * rt_s
    diff = lhs + re_s - rhs
    out_r[...] = jnp.sqrt(jnp.sum(diff * diff, axis=1, keepdims=True) + 1e-12)


def _tc_energy(re, rt, le, lt, rhe, rht, li, ri, hi):
    in_spec = pl.BlockSpec((_BLK, 2 * D), lambda i: (i, 0))
    idx_spec = pl.BlockSpec((_BLK, 1), lambda i: (i, 0))
    out_spec = pl.BlockSpec((_BLK, 1), lambda i: (i, 0))
    return pl.pallas_call(
        _tc_energy_body,
        grid=(BATCH // _BLK,),
        in_specs=[in_spec] * 6 + [idx_spec] * 3,
        out_specs=out_spec,
        out_shape=jax.ShapeDtypeStruct((BATCH, 1), jnp.float32),
    )(re, rt, le, lt, rhe, rht,
      li[:, None], ri[:, None], hi[:, None])


def kernel(triplets, ent_embeds, rel_embeds, ent_transfer, rel_transfer):
    lhs_idx = triplets[:, 0]
    rel_idx = triplets[:, 1]
    rhs_idx = triplets[:, 2]

    def fold(e):
        return jnp.where(e < SPLIT, e, e - SPLIT)

    ee, re, et, rt = _tc_repack(ent_embeds.T, rel_embeds.T,
                                ent_transfer.T, rel_transfer.T)
    o_re, o_rt, o_le, o_lt, o_rhe, o_rht = _sc_gather6(
        fold(lhs_idx), fold(rel_idx), fold(rhs_idx), ee, re, et, rt)
    return _tc_energy(o_re, o_rt, o_le, o_lt, o_rhe, o_rht,
                      lhs_idx, rel_idx, rhs_idx).reshape(BATCH)


# speedup vs baseline: 3.2852x; 1.1782x over previous
import functools

import jax
import jax.numpy as jnp
from jax import lax
from jax.experimental import pallas as pl
from jax.experimental.pallas import tpu as pltpu
from jax.experimental.pallas import tpu_sc as plsc

NUM = 1000000
QS = 249856
TAIL0 = NUM - 2048
ROWS = QS + 2048
BATCH = 16384
D = 64
NC = 2
NS = 16
NW = NC * NS
BPW = BATCH // NW
CH = BPW // 2
TBK = 2048
_QBLK = QS // TBK


def _pack16(x):
    h = x.astype(jnp.bfloat16)
    return lax.bitcast_convert_type(h, jnp.uint16).astype(jnp.uint32)


def _tc_repack_body(*refs):
    i = pl.program_id(0)
    ins, outs = refs[:20], refs[20:]
    for t in range(4):
        q = [ins[5 * t + m] for m in range(4)]
        tail = ins[5 * t + 4]
        o = outs[t]

        @pl.when(i < _QBLK)
        def _(q=q, o=o):
            zt = jnp.concatenate([r[...] for r in q], axis=0).T
            u = _pack16(zt[:, 0:128])
            v = _pack16(zt[:, 128:256])
            o[...] = u | (v << 16)

        @pl.when(i == _QBLK)
        def _(tail=tail, o=o):
            zt = jnp.concatenate([tail[...], tail[...]], axis=0).T
            u = _pack16(zt)
            o[...] = u | (u << 16)


def _tc_repack(tables):
    nblk = ROWS // TBK
    in_q = [pl.BlockSpec(
        (D, TBK),
        lambda i, m=m: (0, jnp.minimum(i, _QBLK - 1) + m * _QBLK))
        for m in range(4)]
    tail_s = pl.BlockSpec((D, TBK), lambda i: (0, 0))
    out_s = pl.BlockSpec((TBK, 2 * D), lambda i: (i, 0))
    args = []
    for t, tl in tables:
        args += [t, t, t, t, tl]
    return pl.pallas_call(
        _tc_repack_body,
        grid=(nblk,),
        in_specs=(in_q + [tail_s]) * 4,
        out_specs=[out_s] * 4,
        out_shape=[jax.ShapeDtypeStruct((ROWS, 2 * D), jnp.uint32)] * 4,
    )(*args)


def _sc_gather6(lhs_idx, rel_idx, rhs_idx, ee, re, et, rt):
    mesh = plsc.VectorSubcoreMesh(core_axis_name="c", subcore_axis_name="s")
    out_t = jax.ShapeDtypeStruct((BATCH, 2 * D), jnp.uint32)

    @functools.partial(
        pl.kernel,
        mesh=mesh,
        out_type=[out_t] * 6,
        scratch_types=[
            pltpu.VMEM((CH,), jnp.int32),
            pltpu.VMEM((CH,), jnp.int32),
            pltpu.VMEM((CH, 2 * D), jnp.uint32),
            pltpu.VMEM((CH, 2 * D), jnp.uint32),
            pltpu.SemaphoreType.DMA,
            pltpu.SemaphoreType.DMA,
            pltpu.SemaphoreType.DMA,
            pltpu.SemaphoreType.DMA,
        ],
    )
    def k(lhs_hbm, rel_hbm, rhs_hbm, ee_hbm, re_hbm, et_hbm, rt_hbm,
          o_re, o_rt, o_le, o_lt, o_rhe, o_rht,
          idx0, idx1, buf0, buf1, g0, g1, w0, w1):
        wid = lax.axis_index("s") * NC + lax.axis_index("c")
        base = wid * BPW

        def job(idx_hbm, tbl, out, load_idx):
            if load_idx:
                pltpu.sync_copy(idx_hbm.at[pl.ds(base, CH)], idx0)
                pltpu.sync_copy(idx_hbm.at[pl.ds(base + CH, CH)], idx1)
            cp0 = pltpu.async_copy(tbl.at[idx0], buf0, g0)
            cp1 = pltpu.async_copy(tbl.at[idx1], buf1, g1)
            cp0.wait()
            wr0 = pltpu.make_async_copy(buf0, out.at[pl.ds(base, CH)], w0)
            wr0.start()
            cp1.wait()
            wr1 = pltpu.make_async_copy(
                buf1, out.at[pl.ds(base + CH, CH)], w1)
            wr1.start()
            wr0.wait()
            wr1.wait()

        job(rel_hbm, re_hbm, o_re, True)
        job(rel_hbm, rt_hbm, o_rt, False)
        job(lhs_hbm, ee_hbm, o_le, True)
        job(lhs_hbm, et_hbm, o_lt, False)
        job(rhs_hbm, ee_hbm, o_rhe, True)
        job(rhs_hbm, et_hbm, o_rht, False)

    return k(lhs_idx, rel_idx, rhs_idx, ee, re, et, rt)


_BLK = 2048


def _tc_energy_body(re_r, rt_r, le_r, lt_r, rhe_r, rht_r,
                    lq_r, rq_r, hq_r, out_r):
    def unpack(ref, q_r):
        x = ref[...]
        q = q_r[...]
        y = jnp.where((q & 1) == 1, x[:, D:2 * D], x[:, 0:D])
        bits = jnp.where(q >= 2, y >> 16, y & jnp.uint32(0xFFFF))
        h = lax.bitcast_convert_type(bits.astype(jnp.uint16), jnp.bfloat16)
        return h.astype(jnp.float32)

    def scaled(x):
        n = jnp.sqrt(jnp.sum(x * x, axis=1, keepdims=True))
        return x * jnp.where(n > 1.0, 1.0 / (n + 1e-7), 1.0)

    re_s = scaled(unpack(re_r, rq_r))
    rt_s = scaled(unpack(rt_r, rq_r))
    le_s = scaled(unpack(le_r, lq_r))
    lt_s = scaled(unpack(lt_r, lq_r))
    rhe_s = scaled(unpack(rhe_r, hq_r))
    rht_s = scaled(unpack(rht_r, hq_r))
    lhs = le_s + jnp.sum(le_s * lt_s, axis=1, keepdims=True) * rt_s
    rhs = rhe_s + jnp.sum(rhe_s * rht_s, axis=1, keepdims=True) * rt_s
    diff = lhs + re_s - rhs
    out_r[...] = jnp.sqrt(jnp.sum(diff * diff, axis=1, keepdims=True) + 1e-12)


def _tc_energy(re, rt, le, lt, rhe, rht, lq, rq, hq):
    in_spec = pl.BlockSpec((_BLK, 2 * D), lambda i: (i, 0))
    q_spec = pl.BlockSpec((_BLK, 1), lambda i: (i, 0))
    out_spec = pl.BlockSpec((_BLK, 1), lambda i: (i, 0))
    return pl.pallas_call(
        _tc_energy_body,
        grid=(BATCH // _BLK,),
        in_specs=[in_spec] * 6 + [q_spec] * 3,
        out_specs=out_spec,
        out_shape=jax.ShapeDtypeStruct((BATCH, 1), jnp.float32),
    )(re, rt, le, lt, rhe, rht,
      lq[:, None], rq[:, None], hq[:, None])


def kernel(triplets, ent_embeds, rel_embeds, ent_transfer, rel_transfer):
    lhs_idx = triplets[:, 0]
    rel_idx = triplets[:, 1]
    rhs_idx = triplets[:, 2]

    def quarter(e):
        return jnp.minimum(e // QS, 3).astype(jnp.uint32)

    def fold(e):
        m = jnp.minimum(e // QS, 3)
        return jnp.where(e >= 4 * QS, e - (TAIL0 - QS), e - m * QS)

    views = [(t.T, t.T[:, TAIL0:])
             for t in (ent_embeds, rel_embeds, ent_transfer, rel_transfer)]
    ee, re, et, rt = _tc_repack(views)
    o_re, o_rt, o_le, o_lt, o_rhe, o_rht = _sc_gather6(
        fold(lhs_idx), fold(rel_idx), fold(rhs_idx), ee, re, et, rt)
    return _tc_energy(o_re, o_rt, o_le, o_lt, o_rhe, o_rht,
                      quarter(lhs_idx), quarter(rel_idx),
                      quarter(rhs_idx)).reshape(BATCH)


# speedup vs baseline: 3.3824x; 1.0296x over previous
import functools

import jax
import jax.numpy as jnp
from jax import lax
from jax.experimental import pallas as pl
from jax.experimental.pallas import tpu as pltpu
from jax.experimental.pallas import tpu_sc as plsc

NUM = 1000000
QS = 249856
TBK = 4096
TAIL0 = NUM - TBK
ROWS = QS + TBK
BATCH = 16384
D = 64
NC = 2
NS = 16
NW = NC * NS
BPW = BATCH // NW
CH = BPW // 2
_QBLK = QS // TBK


def _pack16(x):
    h = x.astype(jnp.bfloat16)
    return lax.bitcast_convert_type(h, jnp.uint16).astype(jnp.uint32)


def _tc_repack_body(*refs):
    i = pl.program_id(0)
    ins, outs = refs[:20], refs[20:]
    for t in range(4):
        q = [ins[5 * t + m] for m in range(4)]
        tail = ins[5 * t + 4]
        o = outs[t]

        @pl.when(i < _QBLK)
        def _(q=q, o=o):
            zt = jnp.concatenate([r[...] for r in q], axis=0).T
            u = _pack16(zt[:, 0:128])
            v = _pack16(zt[:, 128:256])
            o[...] = u | (v << 16)

        @pl.when(i == _QBLK)
        def _(tail=tail, o=o):
            zt = jnp.concatenate([tail[...], tail[...]], axis=0).T
            u = _pack16(zt)
            o[...] = u | (u << 16)


def _tc_repack(tables):
    nblk = ROWS // TBK
    in_q = [pl.BlockSpec(
        (D, TBK),
        lambda i, m=m: (0, jnp.minimum(i, _QBLK - 1) + m * _QBLK))
        for m in range(4)]
    tail_s = pl.BlockSpec((D, TBK), lambda i: (0, 0))
    out_s = pl.BlockSpec((TBK, 2 * D), lambda i: (i, 0))
    args = []
    for t, tl in tables:
        args += [t, t, t, t, tl]
    return pl.pallas_call(
        _tc_repack_body,
        grid=(nblk,),
        in_specs=(in_q + [tail_s]) * 4,
        out_specs=[out_s] * 4,
        out_shape=[jax.ShapeDtypeStruct((ROWS, 2 * D), jnp.uint32)] * 4,
        compiler_params=pltpu.CompilerParams(
            vmem_limit_bytes=128 * 1024 * 1024),
    )(*args)


def _sc_gather6(lhs_idx, rel_idx, rhs_idx, ee, re, et, rt):
    mesh = plsc.VectorSubcoreMesh(core_axis_name="c", subcore_axis_name="s")
    out_t = jax.ShapeDtypeStruct((BATCH, 2 * D), jnp.uint32)

    @functools.partial(
        pl.kernel,
        mesh=mesh,
        out_type=[out_t] * 6,
        scratch_types=[
            pltpu.VMEM((CH,), jnp.int32),
            pltpu.VMEM((CH,), jnp.int32),
            pltpu.VMEM((CH, 2 * D), jnp.uint32),
            pltpu.VMEM((CH, 2 * D), jnp.uint32),
            pltpu.SemaphoreType.DMA,
            pltpu.SemaphoreType.DMA,
            pltpu.SemaphoreType.DMA,
            pltpu.SemaphoreType.DMA,
        ],
    )
    def k(lhs_hbm, rel_hbm, rhs_hbm, ee_hbm, re_hbm, et_hbm, rt_hbm,
          o_re, o_rt, o_le, o_lt, o_rhe, o_rht,
          idx0, idx1, buf0, buf1, g0, g1, w0, w1):
        wid = lax.axis_index("s") * NC + lax.axis_index("c")
        base = wid * BPW

        def job(idx_hbm, tbl, out, load_idx):
            if load_idx:
                pltpu.sync_copy(idx_hbm.at[pl.ds(base, CH)], idx0)
                pltpu.sync_copy(idx_hbm.at[pl.ds(base + CH, CH)], idx1)
            cp0 = pltpu.async_copy(tbl.at[idx0], buf0, g0)
            cp1 = pltpu.async_copy(tbl.at[idx1], buf1, g1)
            cp0.wait()
            wr0 = pltpu.make_async_copy(buf0, out.at[pl.ds(base, CH)], w0)
            wr0.start()
            cp1.wait()
            wr1 = pltpu.make_async_copy(
                buf1, out.at[pl.ds(base + CH, CH)], w1)
            wr1.start()
            wr0.wait()
            wr1.wait()

        job(rel_hbm, re_hbm, o_re, True)
        job(rel_hbm, rt_hbm, o_rt, False)
        job(lhs_hbm, ee_hbm, o_le, True)
        job(lhs_hbm, et_hbm, o_lt, False)
        job(rhs_hbm, ee_hbm, o_rhe, True)
        job(rhs_hbm, et_hbm, o_rht, False)

    return k(lhs_idx, rel_idx, rhs_idx, ee, re, et, rt)


_BLK = 2048


def _tc_energy_body(re_r, rt_r, le_r, lt_r, rhe_r, rht_r,
                    lq_r, rq_r, hq_r, out_r):
    def unpack(ref, q_r):
        x = ref[...]
        q = q_r[...]
        y = jnp.where((q & 1) == 1, x[:, D:2 * D], x[:, 0:D])
        bits = jnp.where(q >= 2, y >> 16, y & jnp.uint32(0xFFFF))
        h = lax.bitcast_convert_type(bits.astype(jnp.uint16), jnp.bfloat16)
        return h.astype(jnp.float32)

    def scaled(x):
        n = jnp.sqrt(jnp.sum(x * x, axis=1, keepdims=True))
        return x * jnp.where(n > 1.0, 1.0 / (n + 1e-7), 1.0)

    re_s = scaled(unpack(re_r, rq_r))
    rt_s = scaled(unpack(rt_r, rq_r))
    le_s = scaled(unpack(le_r, lq_r))
    lt_s = scaled(unpack(lt_r, lq_r))
    rhe_s = scaled(unpack(rhe_r, hq_r))
    rht_s = scaled(unpack(rht_r, hq_r))
    lhs = le_s + jnp.sum(le_s * lt_s, axis=1, keepdims=True) * rt_s
    rhs = rhe_s + jnp.sum(rhe_s * rht_s, axis=1, keepdims=True) * rt_s
    diff = lhs + re_s - rhs
    out_r[...] = jnp.sqrt(jnp.sum(diff * diff, axis=1, keepdims=True) + 1e-12)


def _tc_energy(re, rt, le, lt, rhe, rht, lq, rq, hq):
    in_spec = pl.BlockSpec((_BLK, 2 * D), lambda i: (i, 0))
    q_spec = pl.BlockSpec((_BLK, 1), lambda i: (i, 0))
    out_spec = pl.BlockSpec((_BLK, 1), lambda i: (i, 0))
    return pl.pallas_call(
        _tc_energy_body,
        grid=(BATCH // _BLK,),
        in_specs=[in_spec] * 6 + [q_spec] * 3,
        out_specs=out_spec,
        out_shape=jax.ShapeDtypeStruct((BATCH, 1), jnp.float32),
    )(re, rt, le, lt, rhe, rht,
      lq[:, None], rq[:, None], hq[:, None])


def kernel(triplets, ent_embeds, rel_embeds, ent_transfer, rel_transfer):
    lhs_idx = triplets[:, 0]
    rel_idx = triplets[:, 1]
    rhs_idx = triplets[:, 2]

    def quarter(e):
        return jnp.minimum(e // QS, 3).astype(jnp.uint32)

    def fold(e):
        m = jnp.minimum(e // QS, 3)
        return jnp.where(e >= 4 * QS, e - (TAIL0 - QS), e - m * QS)

    views = [(t.T, t.T[:, TAIL0:])
             for t in (ent_embeds, rel_embeds, ent_transfer, rel_transfer)]
    ee, re, et, rt = _tc_repack(views)
    o_re, o_rt, o_le, o_lt, o_rhe, o_rht = _sc_gather6(
        fold(lhs_idx), fold(rel_idx), fold(rhs_idx), ee, re, et, rt)
    return _tc_energy(o_re, o_rt, o_le, o_lt, o_rhe, o_rht,
                      quarter(lhs_idx), quarter(rel_idx),
                      quarter(rhs_idx)).reshape(BATCH)


# speedup vs baseline: 3.4982x; 1.0342x over previous
import functools

import jax
import jax.numpy as jnp
from jax import lax
from jax.experimental import pallas as pl
from jax.experimental.pallas import tpu as pltpu
from jax.experimental.pallas import tpu_sc as plsc

NUM = 1000000
QS = 249856
TBK = 4096
TAIL0 = NUM - TBK
ROWS = QS + TBK
BATCH = 16384
D = 64
NC = 2
NS = 16
NW = NC * NS
BPW = BATCH // NW
CH = BPW // 2
_QBLK = QS // TBK


def _pack16(x):
    h = x.astype(jnp.bfloat16)
    return lax.bitcast_convert_type(h, jnp.uint16).astype(jnp.uint32)


def _tc_repack_body(*refs):
    i = pl.program_id(0)
    ins, outs = refs[:20], refs[20:]
    for t in range(4):
        q = [ins[5 * t + m] for m in range(4)]
        tail = ins[5 * t + 4]
        o = outs[t]

        @pl.when(i < _QBLK)
        def _(q=q, o=o):
            zt = jnp.concatenate([r[...] for r in q], axis=0).T
            u = _pack16(zt[:, 0:128])
            v = _pack16(zt[:, 128:256])
            o[...] = u | (v << 16)

        @pl.when(i == _QBLK)
        def _(tail=tail, o=o):
            zt = jnp.concatenate([tail[...], tail[...]], axis=0).T
            u = _pack16(zt)
            o[...] = u | (u << 16)


def _tc_repack(tables):
    nblk = ROWS // TBK
    in_q = [pl.BlockSpec(
        (D, TBK),
        lambda i, m=m: (0, jnp.minimum(i, _QBLK - 1) + m * _QBLK))
        for m in range(4)]
    tail_s = pl.BlockSpec((D, TBK), lambda i: (0, 0))
    out_s = pl.BlockSpec((TBK, 2 * D), lambda i: (i, 0))
    args = []
    for t, tl in tables:
        args += [t, t, t, t, tl]
    return pl.pallas_call(
        _tc_repack_body,
        grid=(nblk,),
        in_specs=(in_q + [tail_s]) * 4,
        out_specs=[out_s] * 4,
        out_shape=[jax.ShapeDtypeStruct((ROWS, 2 * D), jnp.uint32)] * 4,
        compiler_params=pltpu.CompilerParams(
            vmem_limit_bytes=128 * 1024 * 1024),
    )(*args)


def _sc_gather6(lhs_idx, rel_idx, rhs_idx, lhs_q, rel_q, rhs_q,
                ee, re, et, rt):
    mesh = plsc.VectorSubcoreMesh(core_axis_name="c", subcore_axis_name="s")
    out_t = jax.ShapeDtypeStruct((BATCH // 2, 2 * D), jnp.uint32)

    @functools.partial(
        pl.kernel,
        mesh=mesh,
        out_type=[out_t] * 6,
        scratch_types=[
            pltpu.VMEM((CH,), jnp.int32),
            pltpu.VMEM((CH,), jnp.int32),
            pltpu.VMEM((CH,), jnp.int32),
            pltpu.VMEM((CH,), jnp.int32),
            pltpu.VMEM((CH, 2 * D), jnp.uint32),
            pltpu.VMEM((CH, 2 * D), jnp.uint32),
            pltpu.VMEM((CH // 2, 2 * D), jnp.uint32),
            pltpu.VMEM((CH // 2, 2 * D), jnp.uint32),
            pltpu.SemaphoreType.DMA,
            pltpu.SemaphoreType.DMA,
            pltpu.SemaphoreType.DMA,
            pltpu.SemaphoreType.DMA,
        ],
    )
    def k(lhs_hbm, rel_hbm, rhs_hbm, lq_hbm, rq_hbm, hq_hbm,
          ee_hbm, re_hbm, et_hbm, rt_hbm,
          o_re, o_rt, o_le, o_lt, o_rhe, o_rht,
          idx0, idx1, qv0, qv1, buf0, buf1, sbuf0, sbuf1, g0, g1, w0, w1):
        wid = lax.axis_index("s") * NC + lax.axis_index("c")
        base = wid * BPW
        obase = wid * (BPW // 2)

        def select(buf, qv, sbuf):
            def body(g, _):
                qvec = qv[pl.ds(g * 16, 16)]
                for j in range(16):
                    r = g * 16 + j
                    off = (qvec[j] & 1) * D
                    dr = g * 8 + j // 2
                    db = (j & 1) * D
                    for m2 in range(4):
                        sbuf[dr, pl.ds(db + 16 * m2, 16)] = (
                            buf[r, pl.ds(off + 16 * m2, 16)])
                return 0

            lax.fori_loop(0, CH // 16, body, 0)

        def job(idx_hbm, q_hbm, tbl, out, load_idx):
            if load_idx:
                pltpu.sync_copy(idx_hbm.at[pl.ds(base, CH)], idx0)
                pltpu.sync_copy(idx_hbm.at[pl.ds(base + CH, CH)], idx1)
                pltpu.sync_copy(q_hbm.at[pl.ds(base, CH)], qv0)
                pltpu.sync_copy(q_hbm.at[pl.ds(base + CH, CH)], qv1)
            cp0 = pltpu.async_copy(tbl.at[idx0], buf0, g0)
            cp1 = pltpu.async_copy(tbl.at[idx1], buf1, g1)
            cp0.wait()
            select(buf0, qv0, sbuf0)
            wr0 = pltpu.make_async_copy(
                sbuf0, out.at[pl.ds(obase, CH // 2)], w0)
            wr0.start()
            cp1.wait()
            select(buf1, qv1, sbuf1)
            wr1 = pltpu.make_async_copy(
                sbuf1, out.at[pl.ds(obase + CH // 2, CH // 2)], w1)
            wr1.start()
            wr0.wait()
            wr1.wait()

        job(rel_hbm, rq_hbm, re_hbm, o_re, True)
        job(rel_hbm, rq_hbm, rt_hbm, o_rt, False)
        job(lhs_hbm, lq_hbm, ee_hbm, o_le, True)
        job(lhs_hbm, lq_hbm, et_hbm, o_lt, False)
        job(rhs_hbm, hq_hbm, ee_hbm, o_rhe, True)
        job(rhs_hbm, hq_hbm, et_hbm, o_rht, False)

    return k(lhs_idx, rel_idx, rhs_idx, lhs_q, rel_q, rhs_q, ee, re, et, rt)


_BLK = 1024


def _tc_energy_body(re_r, rt_r, le_r, lt_r, rhe_r, rht_r,
                    lq_r, rq_r, hq_r, out_r):
    lane = lax.broadcasted_iota(jnp.int32, (_BLK, 2 * D), 1)
    lo_half = lane < D

    def unpack(ref, q_r):
        x = ref[...]
        q = q_r[...]
        qh = jnp.where(lo_half, q[:, 0:1], q[:, 1:2])
        bits = jnp.where(qh >= 2, x & jnp.uint32(0xFFFF0000), x << 16)
        return lax.bitcast_convert_type(bits, jnp.float32)

    re_ = unpack(re_r, rq_r)
    rt_ = unpack(rt_r, rq_r)
    le_ = unpack(le_r, lq_r)
    lt_ = unpack(lt_r, lq_r)
    rhe_ = unpack(rhe_r, hq_r)
    rht_ = unpack(rht_r, hq_r)

    prods = [re_ * re_, rt_ * rt_, le_ * le_, lt_ * lt_,
             rhe_ * rhe_, rht_ * rht_, le_ * lt_, rhe_ * rht_]
    P = jnp.concatenate(prods, axis=1)
    r16 = lax.broadcasted_iota(jnp.int32, (16 * D, 16), 0) // D
    c16 = lax.broadcasted_iota(jnp.int32, (16 * D, 16), 1)
    W = (r16 == c16).astype(jnp.float32)
    S = jax.lax.dot_general(P, W, (((1,), (0,)), ((), ())),
                            preferred_element_type=jnp.float32)

    n = jnp.sqrt(S[:, 0:12])
    s = jnp.where(n > 1.0, 1.0 / (n + 1e-7), 1.0)

    def half_bcast(pair):
        return jnp.where(lo_half, pair[:, 0:1], pair[:, 1:2])

    s_re, s_rt = s[:, 0:2], s[:, 2:4]
    s_le, s_lt = s[:, 4:6], s[:, 6:8]
    s_rhe, s_rht = s[:, 8:10], s[:, 10:12]
    d_l = S[:, 12:14] * s_le * s_lt
    d_r = S[:, 14:16] * s_rhe * s_rht
    coef = (d_l - d_r) * s_rt
    diff = (half_bcast(s_le) * le_ - half_bcast(s_rhe) * rhe_
            + half_bcast(s_re) * re_ + half_bcast(coef) * rt_)
    r2 = lax.broadcasted_iota(jnp.int32, (2 * D, 2), 0) // D
    c2 = lax.broadcasted_iota(jnp.int32, (2 * D, 2), 1)
    W2 = (r2 == c2).astype(jnp.float32)
    e2 = jax.lax.dot_general(diff * diff, W2, (((1,), (0,)), ((), ())),
                             preferred_element_type=jnp.float32)
    out_r[...] = jnp.sqrt(e2 + 1e-12)


def _tc_energy(re, rt, le, lt, rhe, rht, lq, rq, hq):
    in_spec = pl.BlockSpec((_BLK, 2 * D), lambda i: (i, 0))
    q_spec = pl.BlockSpec((_BLK, 2), lambda i: (i, 0))
    out_spec = pl.BlockSpec((_BLK, 2), lambda i: (i, 0))
    return pl.pallas_call(
        _tc_energy_body,
        grid=(BATCH // 2 // _BLK,),
        in_specs=[in_spec] * 6 + [q_spec] * 3,
        out_specs=out_spec,
        out_shape=jax.ShapeDtypeStruct((BATCH // 2, 2), jnp.float32),
    )(re, rt, le, lt, rhe, rht,
      lq.reshape(BATCH // 2, 2), rq.reshape(BATCH // 2, 2),
      hq.reshape(BATCH // 2, 2))


def kernel(triplets, ent_embeds, rel_embeds, ent_transfer, rel_transfer):
    lhs_idx = triplets[:, 0]
    rel_idx = triplets[:, 1]
    rhs_idx = triplets[:, 2]

    def quarter(e):
        return jnp.minimum(e // QS, 3).astype(jnp.uint32)

    def fold(e):
        m = jnp.minimum(e // QS, 3)
        return jnp.where(e >= 4 * QS, e - (TAIL0 - QS), e - m * QS)

    views = [(t.T, t.T[:, TAIL0:])
             for t in (ent_embeds, rel_embeds, ent_transfer, rel_transfer)]
    ee, re, et, rt = _tc_repack(views)
    lq = quarter(lhs_idx).astype(jnp.int32)
    rq = quarter(rel_idx).astype(jnp.int32)
    hq = quarter(rhs_idx).astype(jnp.int32)
    o_re, o_rt, o_le, o_lt, o_rhe, o_rht = _sc_gather6(
        fold(lhs_idx), fold(rel_idx), fold(rhs_idx), lq, rq, hq,
        ee, re, et, rt)
    return _tc_energy(o_re, o_rt, o_le, o_lt, o_rhe, o_rht,
                      lq, rq, hq).reshape(BATCH)
